# baseline (speedup 1.0000x reference)
import functools

import jax
import jax.numpy as jnp
import numpy as np
from jax import lax
from jax.experimental import pallas as pl
from jax.experimental.pallas import tpu as pltpu
from jax.experimental.pallas import tpu_sc as plsc

N_NODES = 10000
N_EDGES = 320000
D_FEAT = 128
D_EDGE = 16
D_OUT = 128
D_HALF = D_OUT // 2

NC = 2
NS = 16
NW = NC * NS
E_PER_W = N_EDGES // NW
CHUNK = 80
N_CHUNKS = E_PER_W // CHUNK

_HI_MASK = np.int32(-65536)


def _round_bits(v):
    return lax.bitcast_convert_type(
        v.astype(jnp.bfloat16).astype(jnp.float32), jnp.int32)


def _pack(ve, vo):
    return lax.shift_right_logical(_round_bits(ve), 16) | (
        _round_bits(vo) & _HI_MASK)


def _tc_body(x_ref, w1e_ref, w1o_ref, w3e_ref, w3o_ref, be_ref, bo_ref,
             eattr_ref, w2e_ref, w2o_ref, p1_ref, p3_ref, ea_ref):
    @pl.when(pl.program_id(0) == 0)
    def _():
        xb = x_ref[...]
        p1_ref[...] = _pack(
            jnp.dot(xb, w1e_ref[...], preferred_element_type=jnp.float32)
            + be_ref[...],
            jnp.dot(xb, w1o_ref[...], preferred_element_type=jnp.float32)
            + bo_ref[...],
        )
        p3_ref[...] = _pack(
            jnp.dot(xb, w3e_ref[...], preferred_element_type=jnp.float32),
            jnp.dot(xb, w3o_ref[...], preferred_element_type=jnp.float32),
        )

    eb = eattr_ref[...]
    ea_ref[...] = _pack(
        jnp.dot(eb, w2e_ref[...], preferred_element_type=jnp.float32),
        jnp.dot(eb, w2o_ref[...], preferred_element_type=jnp.float32),
    )


def _tc_stage(x, w1e, w1o, w3e, w3o, b_e, b_o, edge_attr, w2e, w2o):
    grid = 40
    blk = N_EDGES // grid
    whole = lambda: pl.BlockSpec((D_FEAT, D_HALF), lambda i: (0, 0))
    bias = lambda: pl.BlockSpec((1, D_HALF), lambda i: (0, 0))
    return pl.pallas_call(
        _tc_body,
        grid=(grid,),
        in_specs=[
            pl.BlockSpec((N_NODES, D_FEAT), lambda i: (0, 0)),
            whole(), whole(), whole(), whole(), bias(), bias(),
            pl.BlockSpec((blk, D_EDGE), lambda i: (i, 0)),
            pl.BlockSpec((D_EDGE, D_HALF), lambda i: (0, 0)),
            pl.BlockSpec((D_EDGE, D_HALF), lambda i: (0, 0)),
        ],
        out_specs=[
            pl.BlockSpec((N_NODES, D_HALF), lambda i: (0, 0)),
            pl.BlockSpec((N_NODES, D_HALF), lambda i: (0, 0)),
            pl.BlockSpec((blk, D_HALF), lambda i: (i, 0)),
        ],
        out_shape=[
            jax.ShapeDtypeStruct((N_NODES, D_HALF), jnp.int32),
            jax.ShapeDtypeStruct((N_NODES, D_HALF), jnp.int32),
            jax.ShapeDtypeStruct((N_EDGES, D_HALF), jnp.int32),
        ],
    )(x, w1e, w1o, w3e, w3o, b_e, b_o, edge_attr, w2e, w2o)


def _sc_body(p1_hbm, p3_hbm, src_hbm, dst_hbm, ea_hbm, out_hbm,
             idx1_v, idx3_v,
             g1_0, g1_1, g3_0, g3_1, acc_0, acc_1, ob_0, ob_1,
             gsem0, gsem1, easem0, easem1, osem0, osem1):
    wid = lax.axis_index("s") * NC + lax.axis_index("c")
    w_base = wid * E_PER_W

    g1 = (g1_0, g1_1)
    g3 = (g3_0, g3_1)
    acc = (acc_0, acc_1)
    ob = (ob_0, ob_1)
    gsem = (gsem0, gsem1)
    easem = (easem0, easem1)
    osem = (osem0, osem1)

    pltpu.sync_copy(src_hbm.at[pl.ds(w_base, E_PER_W)], idx1_v)
    pltpu.sync_copy(dst_hbm.at[pl.ds(w_base, E_PER_W)], idx3_v)

    def in_descs(c, b):
        base = w_base + c * CHUNK
        lb = c * CHUNK
        return (
            pltpu.make_async_copy(
                p1_hbm.at[idx1_v.at[pl.ds(lb, CHUNK)]], g1[b], gsem[b]),
            pltpu.make_async_copy(
                p3_hbm.at[idx3_v.at[pl.ds(lb, CHUNK)]], g3[b], gsem[b]),
            pltpu.make_async_copy(
                ea_hbm.at[pl.ds(base, CHUNK)], acc[b], easem[b]),
        )

    def out_desc(c, b):
        base = (w_base + c * CHUNK) * D_OUT
        return pltpu.make_async_copy(
            ob[b], out_hbm.at[pl.ds(base, CHUNK * D_OUT)], osem[b])

    def issue(c, b):
        for d in in_descs(c, b):
            d.start()

    def wait_in(c, b):
        for d in in_descs(c, b):
            d.wait()

    def _lo(w):
        return lax.bitcast_convert_type(w << 16, jnp.float32)

    def _hi(w):
        return lax.bitcast_convert_type(w & _HI_MASK, jnp.float32)

    def compute(b):
        def row_body(r, _):
            rbase = r * D_OUT
            for j in range(4):
                sl = pl.ds(j * 16, 16)
                w1v = g1[b][r, sl]
                w3v = g3[b][r, sl]
                wav = acc[b][r, sl]
                ob[b][pl.ds(rbase + j * 16, 16)] = (
                    _lo(w1v) + _lo(w3v) + _lo(wav))
                ob[b][pl.ds(rbase + D_HALF + j * 16, 16)] = (
                    _hi(w1v) + _hi(w3v) + _hi(wav))
            return 0

        lax.fori_loop(0, CHUNK, row_body, 0)

    def step(c, b, do_wait_out, do_issue_next):
        wait_in(c, b)
        if do_wait_out:
            out_desc(c - 2, b).wait()
        compute(b)
        if do_issue_next:
            issue(c + 2, b)
        out_desc(c, b).start()

    issue(0, 0)
    issue(1, 1)
    step(0, 0, False, True)
    step(1, 1, False, True)

    def pair_body(i, _):
        step(2 * i, 0, True, True)
        step(2 * i + 1, 1, True, True)
        return 0

    lax.fori_loop(1, (N_CHUNKS - 3) // 2, pair_body, 0)

    step(N_CHUNKS - 3, 0, True, True)
    step(N_CHUNKS - 2, 1, True, False)
    step(N_CHUNKS - 1, 0, True, False)
    out_desc(N_CHUNKS - 2, 1).wait()
    out_desc(N_CHUNKS - 1, 0).wait()


def _sc_combine(p1, p3, src, dst, ea):
    mesh = plsc.VectorSubcoreMesh(core_axis_name="c", subcore_axis_name="s")
    inblk = lambda: pltpu.VMEM((CHUNK, D_HALF), jnp.int32)
    outblk = lambda: pltpu.VMEM((CHUNK * D_OUT,), jnp.float32)
    f = functools.partial(
        pl.kernel,
        mesh=mesh,
        compiler_params=pltpu.CompilerParams(use_tc_tiling_on_sc=False),
        out_type=jax.ShapeDtypeStruct((N_EDGES * D_OUT,), jnp.float32),
        scratch_types=[
            pltpu.VMEM((E_PER_W,), jnp.int32),
            pltpu.VMEM((E_PER_W,), jnp.int32),
            inblk(), inblk(), inblk(), inblk(), inblk(), inblk(),
            outblk(), outblk(),
            pltpu.SemaphoreType.DMA,
            pltpu.SemaphoreType.DMA,
            pltpu.SemaphoreType.DMA,
            pltpu.SemaphoreType.DMA,
            pltpu.SemaphoreType.DMA,
            pltpu.SemaphoreType.DMA,
        ],
    )(_sc_body)
    return f(p1, p3, src, dst, ea)


@jax.jit
def kernel(x, edge_attr, edge_index, W, b):
    w1 = W[:D_FEAT]
    w2 = W[D_FEAT:D_FEAT + D_EDGE]
    w3 = W[D_FEAT + D_EDGE:]
    p1, p3, ea = _tc_stage(
        x, w1[:, :D_HALF], w1[:, D_HALF:], w3[:, :D_HALF], w3[:, D_HALF:],
        b[:D_HALF].reshape(1, D_HALF), b[D_HALF:].reshape(1, D_HALF),
        edge_attr, w2[:, :D_HALF], w2[:, D_HALF:])
    out = _sc_combine(p1, p3, edge_index[0], edge_index[1], ea)
    return out.reshape(N_EDGES, D_OUT)
